# baseline (speedup 1.0000x reference)
import functools
import math

import jax
import jax.numpy as jnp
from jax import lax
from jax.experimental import pallas as pl
from jax.experimental.pallas import tpu as pltpu
from jax.experimental.pallas import tpu_sc as plsc

_VOCAB = 100000
_EMB = 128
_SCALE = math.sqrt(float(_EMB))

_NC = 2
_NS = 16
_NW = _NC * _NS

_B = 4096 * 200
_B_PER_W = _B // _NW
_CHUNK = 64
_N_CHUNKS = _B_PER_W // _CHUNK
_K = 5
_NPHASE = _N_CHUNKS // _K


def _scale_body(t_ref, o_ref):
    o_ref[...] = t_ref[...] * _SCALE


_scale_table = pl.pallas_call(
    _scale_body,
    grid=(100,),
    in_specs=[pl.BlockSpec((_VOCAB // 100, _EMB), lambda i: (i, 0))],
    out_specs=pl.BlockSpec((_VOCAB // 100, _EMB), lambda i: (i, 0)),
    out_shape=jax.ShapeDtypeStruct((_VOCAB, _EMB), jnp.float32),
)


_mesh = plsc.VectorSubcoreMesh(
    core_axis_name="c", subcore_axis_name="s", num_cores=_NC, num_subcores=_NS
)


@functools.partial(
    pl.kernel,
    mesh=_mesh,
    out_type=jax.ShapeDtypeStruct((_B, _EMB), jnp.float32),
    scratch_types=[
        pltpu.VMEM((_B_PER_W,), jnp.int32),
        pltpu.VMEM((2, _K, _CHUNK, _EMB), jnp.float32),
        pltpu.SemaphoreType.DMA,
        pltpu.SemaphoreType.DMA,
        pltpu.SemaphoreType.DMA,
        pltpu.SemaphoreType.DMA,
    ],
)
def _sc_gather(tokens_hbm, table_hbm, out_hbm, idx_v, rows_v,
               gsem0, gsem1, ssem0, ssem1):
    gsems = (gsem0, gsem1)
    ssems = (ssem0, ssem1)
    wid = lax.axis_index("s") * _NC + lax.axis_index("c")
    base = wid * _B_PER_W

    pltpu.sync_copy(tokens_hbm.at[pl.ds(base, _B_PER_W)], idx_v)

    def fire_gathers(pool, cb):
        for j in range(_K):
            isl = idx_v.at[pl.ds((cb + j) * _CHUNK, _CHUNK)]
            pltpu.make_async_copy(
                table_hbm.at[isl], rows_v.at[pool, j], gsems[pool]).start()

    def drain_gathers(pool):
        for j in range(_K):
            pltpu.make_async_copy(
                table_hbm.at[idx_v.at[pl.ds(0, _CHUNK)]],
                rows_v.at[pool, j], gsems[pool]).wait()

    def fire_scatters(pool, cb):
        for j in range(_K):
            dst = out_hbm.at[pl.ds(base + (cb + j) * _CHUNK, _CHUNK)]
            pltpu.make_async_copy(rows_v.at[pool, j], dst, ssems[pool]).start()

    def drain_scatters(pool):
        for j in range(_K):
            pltpu.make_async_copy(
                rows_v.at[pool, j],
                out_hbm.at[pl.ds(base, _CHUNK)], ssems[pool]).wait()

    def phase(p, pool, first, last):
        cb = p * _K
        drain_gathers(pool)
        fire_scatters(pool, cb)
        if not first:
            drain_scatters(1 - pool)
        if not last:
            fire_gathers(1 - pool, cb + _K)

    fire_gathers(0, 0)
    phase(0, 0, True, False)

    def body(i, carry):
        p = 1 + i * 2
        phase(p, 1, False, False)
        phase(p + 1, 0, False, False)
        return carry

    lax.fori_loop(0, (_NPHASE - 2) // 2, body, 0)
    phase(_NPHASE - 1, 1, False, True)
    drain_scatters(1)


def kernel(tokens, table):
    flat = tokens.reshape(-1).astype(jnp.int32)
    scaled = _scale_table(table)
    out = _sc_gather(flat, scaled)
    return out.reshape(tokens.shape + (_EMB,))


# speedup vs baseline: 1.0223x; 1.0223x over previous
import functools
import math

import jax
import jax.numpy as jnp
from jax import lax
from jax.experimental import pallas as pl
from jax.experimental.pallas import tpu as pltpu
from jax.experimental.pallas import tpu_sc as plsc

_VOCAB = 100000
_EMB = 128
_SCALE = math.sqrt(float(_EMB))

_NC = 2
_NS = 16
_NW = _NC * _NS

_B = 4096 * 200
_B_PER_W = _B // _NW
_CHUNK = 64
_N_CHUNKS = _B_PER_W // _CHUNK
_K = 5
_NPHASE = _N_CHUNKS // _K


_mesh = plsc.VectorSubcoreMesh(
    core_axis_name="c", subcore_axis_name="s", num_cores=_NC, num_subcores=_NS
)


@functools.partial(
    pl.kernel,
    mesh=_mesh,
    out_type=jax.ShapeDtypeStruct((_B, _EMB), jnp.float32),
    scratch_types=[
        pltpu.VMEM((_B_PER_W,), jnp.int32),
        pltpu.VMEM((2, _K, _CHUNK, _EMB), jnp.float32),
        pltpu.SemaphoreType.DMA,
        pltpu.SemaphoreType.DMA,
        pltpu.SemaphoreType.DMA,
        pltpu.SemaphoreType.DMA,
    ],
)
def _sc_gather(tokens_hbm, table_hbm, out_hbm, idx_v, rows_v,
               gsem0, gsem1, ssem0, ssem1):
    gsems = (gsem0, gsem1)
    ssems = (ssem0, ssem1)
    wid = lax.axis_index("s") * _NC + lax.axis_index("c")
    base = wid * _B_PER_W

    pltpu.sync_copy(tokens_hbm.at[pl.ds(base, _B_PER_W)], idx_v)

    def fire_gathers(pool, cb):
        for j in range(_K):
            isl = idx_v.at[pl.ds((cb + j) * _CHUNK, _CHUNK)]
            pltpu.make_async_copy(
                table_hbm.at[isl], rows_v.at[pool, j], gsems[pool]).start()

    def drain_gathers(pool):
        for j in range(_K):
            pltpu.make_async_copy(
                table_hbm.at[idx_v.at[pl.ds(0, _CHUNK)]],
                rows_v.at[pool, j], gsems[pool]).wait()

    def scale_pool(pool):
        @plsc.parallel_loop(0, _K * _CHUNK, 1, unroll=2)
        def _(r):
            j = r // _CHUNK
            i = r % _CHUNK
            for t in range(_EMB // 16):
                sl = pl.ds(t * 16, 16)
                rows_v[pool, j, i, sl] = rows_v[pool, j, i, sl] * _SCALE

    def fire_scatters(pool, cb):
        for j in range(_K):
            dst = out_hbm.at[pl.ds(base + (cb + j) * _CHUNK, _CHUNK)]
            pltpu.make_async_copy(rows_v.at[pool, j], dst, ssems[pool]).start()

    def drain_scatters(pool):
        for j in range(_K):
            pltpu.make_async_copy(
                rows_v.at[pool, j],
                out_hbm.at[pl.ds(base, _CHUNK)], ssems[pool]).wait()

    def phase(p, pool, first, last):
        cb = p * _K
        drain_gathers(pool)
        scale_pool(pool)
        fire_scatters(pool, cb)
        if not first:
            drain_scatters(1 - pool)
        if not last:
            fire_gathers(1 - pool, cb + _K)

    fire_gathers(0, 0)
    phase(0, 0, True, False)

    def body(i, carry):
        p = 1 + i * 2
        phase(p, 1, False, False)
        phase(p + 1, 0, False, False)
        return carry

    lax.fori_loop(0, (_NPHASE - 2) // 2, body, 0)
    phase(_NPHASE - 1, 1, False, True)
    drain_scatters(1)


def kernel(tokens, table):
    flat = tokens.reshape(-1).astype(jnp.int32)
    out = _sc_gather(flat, table)
    return out.reshape(tokens.shape + (_EMB,))


# speedup vs baseline: 1.2155x; 1.1890x over previous
import functools
import math

import jax
import jax.numpy as jnp
from jax import lax
from jax.experimental import pallas as pl
from jax.experimental.pallas import tpu as pltpu
from jax.experimental.pallas import tpu_sc as plsc

_VOCAB = 100000
_EMB = 128
_SCALE = math.sqrt(float(_EMB))

_NC = 2
_NS = 16
_NW = _NC * _NS

_B = 4096 * 200
_B_PER_W = _B // _NW
_CHUNK = 64
_N_CHUNKS = _B_PER_W // _CHUNK
_K = 5
_NPHASE = _N_CHUNKS // _K


_mesh = plsc.VectorSubcoreMesh(
    core_axis_name="c", subcore_axis_name="s", num_cores=_NC, num_subcores=_NS
)


@functools.partial(
    pl.kernel,
    mesh=_mesh,
    out_type=jax.ShapeDtypeStruct((_B, _EMB), jnp.float32),
    scratch_types=[
        pltpu.VMEM((_B_PER_W,), jnp.int32),
        pltpu.VMEM((2, _K, _CHUNK, _EMB), jnp.float32),
        pltpu.SemaphoreType.DMA,
        pltpu.SemaphoreType.DMA,
        pltpu.SemaphoreType.DMA,
        pltpu.SemaphoreType.DMA,
    ],
)
def _sc_gather(tokens_hbm, table_hbm, out_hbm, idx_v, rows_v,
               gsem0, gsem1, ssem0, ssem1):
    gsems = (gsem0, gsem1)
    ssems = (ssem0, ssem1)
    wid = lax.axis_index("s") * _NC + lax.axis_index("c")
    base = wid * _B_PER_W

    pltpu.sync_copy(tokens_hbm.at[pl.ds(base, _B_PER_W)], idx_v)

    def fire_gathers(pool, cb):
        for j in range(_K):
            isl = idx_v.at[pl.ds((cb + j) * _CHUNK, _CHUNK)]
            pltpu.make_async_copy(
                table_hbm.at[isl], rows_v.at[pool, j], gsems[pool]).start()

    def drain_gathers(pool):
        for j in range(_K):
            pltpu.make_async_copy(
                table_hbm.at[idx_v.at[pl.ds(0, _CHUNK)]],
                rows_v.at[pool, j], gsems[pool]).wait()

    def scale_pool(pool):
        @plsc.parallel_loop(0, _K * _CHUNK, 1, unroll=4)
        def _(r):
            j = r // _CHUNK
            i = r % _CHUNK
            for t in range(_EMB // 16):
                sl = pl.ds(t * 16, 16)
                rows_v[pool, j, i, sl] = rows_v[pool, j, i, sl] * _SCALE

    def fire_scatters(pool, cb):
        for j in range(_K):
            dst = out_hbm.at[pl.ds(base + (cb + j) * _CHUNK, _CHUNK)]
            pltpu.make_async_copy(rows_v.at[pool, j], dst, ssems[pool]).start()

    def drain_scatters(pool):
        for j in range(_K):
            pltpu.make_async_copy(
                rows_v.at[pool, j],
                out_hbm.at[pl.ds(base, _CHUNK)], ssems[pool]).wait()

    def phase(p, pool, first, last):
        cb = p * _K
        drain_gathers(pool)
        if not first:
            drain_scatters(1 - pool)
        if not last:
            fire_gathers(1 - pool, cb + _K)
        scale_pool(pool)
        fire_scatters(pool, cb)

    fire_gathers(0, 0)
    phase(0, 0, True, False)

    def body(i, carry):
        p = 1 + i * 2
        phase(p, 1, False, False)
        phase(p + 1, 0, False, False)
        return carry

    lax.fori_loop(0, (_NPHASE - 2) // 2, body, 0)
    phase(_NPHASE - 1, 1, False, True)
    drain_scatters(1)


def kernel(tokens, table):
    flat = tokens.reshape(-1).astype(jnp.int32)
    out = _sc_gather(flat, table)
    return out.reshape(tokens.shape + (_EMB,))


# speedup vs baseline: 1.2285x; 1.0107x over previous
import functools
import math

import jax
import jax.numpy as jnp
from jax import lax
from jax.experimental import pallas as pl
from jax.experimental.pallas import tpu as pltpu
from jax.experimental.pallas import tpu_sc as plsc

_VOCAB = 100000
_EMB = 128
_SCALE = math.sqrt(float(_EMB))

_NC = 2
_NS = 16
_NW = _NC * _NS

_B = 4096 * 200
_B_PER_W = _B // _NW
_CHUNK = 64
_N_CHUNKS = _B_PER_W // _CHUNK
_K = 5
_NPHASE = _N_CHUNKS // _K


_mesh = plsc.VectorSubcoreMesh(
    core_axis_name="c", subcore_axis_name="s", num_cores=_NC, num_subcores=_NS
)


@functools.partial(
    pl.kernel,
    mesh=_mesh,
    out_type=jax.ShapeDtypeStruct((_B, _EMB), jnp.float32),
    scratch_types=[
        pltpu.VMEM((_B_PER_W,), jnp.int32),
        pltpu.VMEM((2, _K, _CHUNK, _EMB), jnp.float32),
        pltpu.SemaphoreType.DMA,
        pltpu.SemaphoreType.DMA,
        pltpu.SemaphoreType.DMA,
        pltpu.SemaphoreType.DMA,
    ],
)
def _sc_gather(tokens_hbm, table_hbm, out_hbm, idx_v, rows_v,
               gsem0, gsem1, ssem0, ssem1):
    gsems = (gsem0, gsem1)
    ssems = (ssem0, ssem1)
    wid = lax.axis_index("s") * _NC + lax.axis_index("c")
    base = wid * _B_PER_W

    pltpu.sync_copy(tokens_hbm.at[pl.ds(base, _B_PER_W)], idx_v)

    def fire_gathers(pool, cb):
        for j in range(_K):
            isl = idx_v.at[pl.ds((cb + j) * _CHUNK, _CHUNK)]
            pltpu.make_async_copy(
                table_hbm.at[isl], rows_v.at[pool, j], gsems[pool]).start()

    def wait_gather(pool, j):
        pltpu.make_async_copy(
            table_hbm.at[idx_v.at[pl.ds(0, _CHUNK)]],
            rows_v.at[pool, j], gsems[pool]).wait()

    def scale_chunk(pool, j):
        @plsc.parallel_loop(0, _CHUNK, 1, unroll=4)
        def _(i):
            for t in range(_EMB // 16):
                sl = pl.ds(t * 16, 16)
                rows_v[pool, j, i, sl] = rows_v[pool, j, i, sl] * _SCALE

    def fire_scatter(pool, j, cb):
        dst = out_hbm.at[pl.ds(base + (cb + j) * _CHUNK, _CHUNK)]
        pltpu.make_async_copy(rows_v.at[pool, j], dst, ssems[pool]).start()

    def wait_scatter(pool, j):
        pltpu.make_async_copy(
            rows_v.at[pool, j],
            out_hbm.at[pl.ds(base, _CHUNK)], ssems[pool]).wait()

    def drain_scatters(pool):
        for j in range(_K):
            wait_scatter(pool, j)

    def phase(p, pool, first, last):
        cb = p * _K
        for j in range(_K):
            wait_gather(pool, j)
            scale_chunk(pool, j)
            fire_scatter(pool, j, cb)
        for j in range(_K):
            if not first:
                wait_scatter(1 - pool, j)
            if not last:
                isl = idx_v.at[pl.ds((cb + _K + j) * _CHUNK, _CHUNK)]
                pltpu.make_async_copy(
                    table_hbm.at[isl], rows_v.at[1 - pool, j],
                    gsems[1 - pool]).start()

    fire_gathers(0, 0)
    phase(0, 0, True, False)

    def body(i, carry):
        p = 1 + i * 2
        phase(p, 1, False, False)
        phase(p + 1, 0, False, False)
        return carry

    lax.fori_loop(0, (_NPHASE - 2) // 2, body, 0)
    phase(_NPHASE - 1, 1, False, True)
    drain_scatters(1)


def kernel(tokens, table):
    flat = tokens.reshape(-1).astype(jnp.int32)
    out = _sc_gather(flat, table)
    return out.reshape(tokens.shape + (_EMB,))


# speedup vs baseline: 1.2332x; 1.0038x over previous
import functools
import math

import jax
import jax.numpy as jnp
from jax import lax
from jax.experimental import pallas as pl
from jax.experimental.pallas import tpu as pltpu
from jax.experimental.pallas import tpu_sc as plsc

_VOCAB = 100000
_EMB = 128
_SCALE = math.sqrt(float(_EMB))

_NC = 2
_NS = 16
_NW = _NC * _NS

_B = 4096 * 200
_B_PER_W = _B // _NW
_CHUNK = 40
_N_CHUNKS = _B_PER_W // _CHUNK
_K = 8
_NPHASE = _N_CHUNKS // _K


_mesh = plsc.VectorSubcoreMesh(
    core_axis_name="c", subcore_axis_name="s", num_cores=_NC, num_subcores=_NS
)


@functools.partial(
    pl.kernel,
    mesh=_mesh,
    out_type=jax.ShapeDtypeStruct((_B, _EMB), jnp.float32),
    scratch_types=[
        pltpu.VMEM((_B_PER_W,), jnp.int32),
        pltpu.VMEM((2, _K, _CHUNK, _EMB), jnp.float32),
        pltpu.SemaphoreType.DMA,
        pltpu.SemaphoreType.DMA,
        pltpu.SemaphoreType.DMA,
        pltpu.SemaphoreType.DMA,
    ],
)
def _sc_gather(tokens_hbm, table_hbm, out_hbm, idx_v, rows_v,
               gsem0, gsem1, ssem0, ssem1):
    gsems = (gsem0, gsem1)
    ssems = (ssem0, ssem1)
    wid = lax.axis_index("s") * _NC + lax.axis_index("c")
    base = wid * _B_PER_W

    pltpu.sync_copy(tokens_hbm.at[pl.ds(base, _B_PER_W)], idx_v)

    def fire_gathers(pool, cb):
        for j in range(_K):
            isl = idx_v.at[pl.ds((cb + j) * _CHUNK, _CHUNK)]
            pltpu.make_async_copy(
                table_hbm.at[isl], rows_v.at[pool, j], gsems[pool]).start()

    def wait_gather(pool, j):
        pltpu.make_async_copy(
            table_hbm.at[idx_v.at[pl.ds(0, _CHUNK)]],
            rows_v.at[pool, j], gsems[pool]).wait()

    def scale_chunk(pool, j):
        @plsc.parallel_loop(0, _CHUNK, 1, unroll=4)
        def _(i):
            for t in range(_EMB // 16):
                sl = pl.ds(t * 16, 16)
                rows_v[pool, j, i, sl] = rows_v[pool, j, i, sl] * _SCALE

    def fire_scatter(pool, j, cb):
        dst = out_hbm.at[pl.ds(base + (cb + j) * _CHUNK, _CHUNK)]
        pltpu.make_async_copy(rows_v.at[pool, j], dst, ssems[pool]).start()

    def wait_scatter(pool, j):
        pltpu.make_async_copy(
            rows_v.at[pool, j],
            out_hbm.at[pl.ds(base, _CHUNK)], ssems[pool]).wait()

    def drain_scatters(pool):
        for j in range(_K):
            wait_scatter(pool, j)

    def phase(p, pool, first, last):
        cb = p * _K
        for j in range(_K):
            wait_gather(pool, j)
            scale_chunk(pool, j)
            fire_scatter(pool, j, cb)
        for j in range(_K):
            if not first:
                wait_scatter(1 - pool, j)
            if not last:
                isl = idx_v.at[pl.ds((cb + _K + j) * _CHUNK, _CHUNK)]
                pltpu.make_async_copy(
                    table_hbm.at[isl], rows_v.at[1 - pool, j],
                    gsems[1 - pool]).start()

    fire_gathers(0, 0)
    phase(0, 0, True, False)

    def body(i, carry):
        p = 1 + i * 2
        phase(p, 1, False, False)
        phase(p + 1, 0, False, False)
        return carry

    lax.fori_loop(0, (_NPHASE - 2) // 2, body, 0)
    phase(_NPHASE - 1, 1, False, True)
    drain_scatters(1)


def kernel(tokens, table):
    flat = tokens.reshape(-1).astype(jnp.int32)
    out = _sc_gather(flat, table)
    return out.reshape(tokens.shape + (_EMB,))
